# baseline (speedup 1.0000x reference)
import jax
import jax.numpy as jnp
from jax.experimental import pallas as pl
from jax.experimental.pallas import tpu as pltpu

_NUM_EXPERTS = 64
_CAPACITY = 320.0
_S_BLK = 1024
_N_SPLIT = 2
_S_SUB = _S_BLK // _N_SPLIT
_C_CHUNK = 512


def _router_kernel(*refs):
    x_refs = refs[:_N_SPLIT]
    w_ref, ei_ref, rp_ref, lg_ref, carry_ref = refs[_N_SPLIT:]
    j = pl.program_id(1)

    @pl.when(j == 0)
    def _():
        carry_ref[...] = jnp.zeros_like(carry_ref)

    w = w_ref[...]
    dims = (((1,), (1,)), ((), ()))
    logits = jnp.concatenate(
        [jax.lax.dot_general(xr[0], w, dims,
                             preferred_element_type=jnp.float32)
         for xr in x_refs],
        axis=0,
    )
    lg_ref[0] = logits

    m = jnp.max(logits, axis=-1, keepdims=True)
    s = jnp.sum(jnp.exp(logits - m), axis=-1, keepdims=True)
    top_val = 1.0 / s

    eidx = jax.lax.broadcasted_iota(jnp.int32, logits.shape, 1)
    amin = jnp.min(
        jnp.where(logits == m, eidx, _NUM_EXPERTS), axis=-1, keepdims=True
    )
    onehot = (eidx == amin).astype(jnp.float32)

    ri = jax.lax.broadcasted_iota(jnp.int32, (_C_CHUNK, _C_CHUNK), 0)
    ci = jax.lax.broadcasted_iota(jnp.int32, (_C_CHUNK, _C_CHUNK), 1)
    tril = (ci <= ri).astype(jnp.bfloat16)
    offset = carry_ref[...]
    parts = []
    for g in range(_S_BLK // _C_CHUNK):
        oh = onehot[g * _C_CHUNK:(g + 1) * _C_CHUNK, :].astype(jnp.bfloat16)
        cs = jax.lax.dot_general(
            tril, oh, (((1,), (0,)), ((), ())),
            preferred_element_type=jnp.float32,
        ) + offset
        offset = cs[_C_CHUNK - 1:_C_CHUNK, :]
        parts.append(cs)
    priority = jnp.concatenate(parts, axis=0)
    carry_ref[...] = offset

    chosen = onehot * (priority <= _CAPACITY).astype(jnp.float32)
    ei_ref[0] = chosen.astype(jnp.int32)
    rp_ref[0] = top_val * chosen


def kernel(x, W):
    B, S, D = x.shape
    E = W.shape[0]
    grid = (B, S // _S_BLK)
    out_shape = (
        jax.ShapeDtypeStruct((B, S, E), jnp.int32),
        jax.ShapeDtypeStruct((B, S, E), jnp.float32),
        jax.ShapeDtypeStruct((B, S, E), jnp.float32),
    )
    out_spec = pl.BlockSpec((1, _S_BLK, E), lambda b, j: (b, j, 0))

    def sub_spec(k):
        return pl.BlockSpec(
            (1, _S_SUB, D), lambda b, j, k=k: (b, _N_SPLIT * j + k, 0)
        )

    return pl.pallas_call(
        _router_kernel,
        grid=grid,
        in_specs=[sub_spec(k) for k in range(_N_SPLIT)]
        + [pl.BlockSpec((E, D), lambda b, j: (0, 0))],
        out_specs=(out_spec, out_spec, out_spec),
        out_shape=out_shape,
        scratch_shapes=[pltpu.VMEM((1, E), jnp.float32)],
    )(*([x] * _N_SPLIT), W)


# speedup vs baseline: 1.0454x; 1.0454x over previous
import jax
import jax.numpy as jnp
from jax.experimental import pallas as pl
from jax.experimental.pallas import tpu as pltpu

_NUM_EXPERTS = 64
_CAPACITY = 320.0
_S_BLK = 2048
_N_SPLIT = 4
_S_SUB = _S_BLK // _N_SPLIT
_C_CHUNK = 512


def _router_kernel(*refs):
    x_refs = refs[:_N_SPLIT]
    w_ref, ei_ref, rp_ref, lg_ref, carry_ref = refs[_N_SPLIT:]
    j = pl.program_id(1)

    @pl.when(j == 0)
    def _():
        carry_ref[...] = jnp.zeros_like(carry_ref)

    w = w_ref[...]
    dims = (((1,), (1,)), ((), ()))
    logits = jnp.concatenate(
        [jax.lax.dot_general(xr[0], w, dims,
                             preferred_element_type=jnp.float32)
         for xr in x_refs],
        axis=0,
    )
    lg_ref[0] = logits

    m = jnp.max(logits, axis=-1, keepdims=True)
    s = jnp.sum(jnp.exp(logits - m), axis=-1, keepdims=True)
    top_val = 1.0 / s

    eidx = jax.lax.broadcasted_iota(jnp.int32, logits.shape, 1)
    amin = jnp.min(
        jnp.where(logits == m, eidx, _NUM_EXPERTS), axis=-1, keepdims=True
    )
    onehot = (eidx == amin).astype(jnp.float32)

    ri = jax.lax.broadcasted_iota(jnp.int32, (_C_CHUNK, _C_CHUNK), 0)
    ci = jax.lax.broadcasted_iota(jnp.int32, (_C_CHUNK, _C_CHUNK), 1)
    tril = (ci <= ri).astype(jnp.bfloat16)
    offset = carry_ref[...]
    parts = []
    for g in range(_S_BLK // _C_CHUNK):
        oh = onehot[g * _C_CHUNK:(g + 1) * _C_CHUNK, :].astype(jnp.bfloat16)
        cs = jax.lax.dot_general(
            tril, oh, (((1,), (0,)), ((), ())),
            preferred_element_type=jnp.float32,
        ) + offset
        offset = cs[_C_CHUNK - 1:_C_CHUNK, :]
        parts.append(cs)
    priority = jnp.concatenate(parts, axis=0)
    carry_ref[...] = offset

    chosen = onehot * (priority <= _CAPACITY).astype(jnp.float32)
    ei_ref[0] = chosen.astype(jnp.int32)
    rp_ref[0] = top_val * chosen


def kernel(x, W):
    B, S, D = x.shape
    E = W.shape[0]
    grid = (B, S // _S_BLK)
    out_shape = (
        jax.ShapeDtypeStruct((B, S, E), jnp.int32),
        jax.ShapeDtypeStruct((B, S, E), jnp.float32),
        jax.ShapeDtypeStruct((B, S, E), jnp.float32),
    )
    out_spec = pl.BlockSpec((1, _S_BLK, E), lambda b, j: (b, j, 0))

    def sub_spec(k):
        return pl.BlockSpec(
            (1, _S_SUB, D), lambda b, j, k=k: (b, _N_SPLIT * j + k, 0)
        )

    return pl.pallas_call(
        _router_kernel,
        grid=grid,
        in_specs=[sub_spec(k) for k in range(_N_SPLIT)]
        + [pl.BlockSpec((E, D), lambda b, j: (0, 0))],
        out_specs=(out_spec, out_spec, out_spec),
        out_shape=out_shape,
        scratch_shapes=[pltpu.VMEM((1, E), jnp.float32)],
        compiler_params=pltpu.CompilerParams(
            dimension_semantics=("parallel", "arbitrary"),
        ),
    )(*([x] * _N_SPLIT), W)
